# baseline (speedup 1.0000x reference)
import jax
import jax.numpy as jnp
from jax.experimental import pallas as pl
from jax.experimental.pallas import tpu as pltpu

_TILE_H = 8


def _fused_heads_body(x_hbm, wc_ref, bc_ref, wr_ref, br_ref, wd_ref, bd_ref,
                      cls_ref, reg_ref, dir_ref, buf, sems):
    B = x_hbm.shape[0]
    n_h = x_hbm.shape[2] // _TILE_H
    n_steps = B * n_h
    g = pl.program_id(0)

    def copy_for(step, slot):
        return pltpu.make_async_copy(
            x_hbm.at[step // n_h, :, pl.ds((step % n_h) * _TILE_H, _TILE_H), :],
            buf.at[slot],
            sems.at[slot],
        )

    @pl.when(g == 0)
    def _prime():
        copy_for(0, 0).start()

    @pl.when(g + 1 < n_steps)
    def _prefetch():
        copy_for(g + 1, (g + 1) % 2).start()

    copy_for(g, g % 2).wait()

    slot = g % 2
    for h in range(_TILE_H):
        xb = buf[slot, :, h, :]
        cls_ref[0, :, h, :] = (
            jnp.dot(wc_ref[...], xb, preferred_element_type=jnp.float32)
            + bc_ref[...]
        )
        reg_ref[0, :, h, :] = (
            jnp.dot(wr_ref[...], xb, preferred_element_type=jnp.float32)
            + br_ref[...]
        )
        dir_ref[0, :, h, :] = (
            jnp.dot(wd_ref[...], xb, preferred_element_type=jnp.float32)
            + bd_ref[...]
        )


def kernel(x, W_cls, b_cls, W_reg, b_reg, W_dir, b_dir):
    B, C, H, W = x.shape
    O_cls = W_cls.shape[0]
    O_reg = W_reg.shape[0]
    O_dir = W_dir.shape[0]
    n_h = H // _TILE_H

    def const_map(g):
        return (0, 0)

    def out_map(g):
        return (g // n_h, 0, g % n_h, 0)

    outs = pl.pallas_call(
        _fused_heads_body,
        grid=(B * n_h,),
        in_specs=[
            pl.BlockSpec(memory_space=pl.ANY),
            pl.BlockSpec((O_cls, C), const_map),
            pl.BlockSpec((O_cls, 1), const_map),
            pl.BlockSpec((O_reg, C), const_map),
            pl.BlockSpec((O_reg, 1), const_map),
            pl.BlockSpec((O_dir, C), const_map),
            pl.BlockSpec((O_dir, 1), const_map),
        ],
        out_specs=[
            pl.BlockSpec((1, O_cls, _TILE_H, W), out_map),
            pl.BlockSpec((1, O_reg, _TILE_H, W), out_map),
            pl.BlockSpec((1, O_dir, _TILE_H, W), out_map),
        ],
        out_shape=[
            jax.ShapeDtypeStruct((B, O_cls, H, W), jnp.float32),
            jax.ShapeDtypeStruct((B, O_reg, H, W), jnp.float32),
            jax.ShapeDtypeStruct((B, O_dir, H, W), jnp.float32),
        ],
        scratch_shapes=[
            pltpu.VMEM((2, C, _TILE_H, W), jnp.float32),
            pltpu.SemaphoreType.DMA((2,)),
        ],
    )(
        x,
        W_cls, b_cls.reshape(O_cls, 1),
        W_reg, b_reg.reshape(O_reg, 1),
        W_dir, b_dir.reshape(O_dir, 1),
    )
    return outs


# speedup vs baseline: 1.0634x; 1.0634x over previous
import jax
import jax.numpy as jnp
from jax.experimental import pallas as pl
from jax.experimental.pallas import tpu as pltpu

_TILE_H = 24


def _fused_heads_body(x_ref, wc_ref, bc_ref, wr_ref, br_ref, wd_ref, bd_ref,
                      cls_ref, reg_ref, dir_ref):
    for h in range(_TILE_H):
        xb = x_ref[0, :, h, :]
        cls_ref[0, :, h, :] = (
            jnp.dot(wc_ref[...], xb, preferred_element_type=jnp.float32)
            + bc_ref[...]
        )
        reg_ref[0, :, h, :] = (
            jnp.dot(wr_ref[...], xb, preferred_element_type=jnp.float32)
            + br_ref[...]
        )
        dir_ref[0, :, h, :] = (
            jnp.dot(wd_ref[...], xb, preferred_element_type=jnp.float32)
            + bd_ref[...]
        )


def kernel(x, W_cls, b_cls, W_reg, b_reg, W_dir, b_dir):
    B, C, H, W = x.shape
    O_cls = W_cls.shape[0]
    O_reg = W_reg.shape[0]
    O_dir = W_dir.shape[0]

    def x_map(b, h):
        return (b, 0, h, 0)

    def const_map(b, h):
        return (0, 0)

    def out_map(b, h):
        return (b, 0, h, 0)

    outs = pl.pallas_call(
        _fused_heads_body,
        grid=(B, pl.cdiv(H, _TILE_H)),
        compiler_params=pltpu.CompilerParams(
            dimension_semantics=(pltpu.PARALLEL, pltpu.PARALLEL),
        ),
        in_specs=[
            pl.BlockSpec((1, C, _TILE_H, W), x_map),
            pl.BlockSpec((O_cls, C), const_map),
            pl.BlockSpec((O_cls, 1), const_map),
            pl.BlockSpec((O_reg, C), const_map),
            pl.BlockSpec((O_reg, 1), const_map),
            pl.BlockSpec((O_dir, C), const_map),
            pl.BlockSpec((O_dir, 1), const_map),
        ],
        out_specs=[
            pl.BlockSpec((1, O_cls, _TILE_H, W), out_map),
            pl.BlockSpec((1, O_reg, _TILE_H, W), out_map),
            pl.BlockSpec((1, O_dir, _TILE_H, W), out_map),
        ],
        out_shape=[
            jax.ShapeDtypeStruct((B, O_cls, H, W), jnp.float32),
            jax.ShapeDtypeStruct((B, O_reg, H, W), jnp.float32),
            jax.ShapeDtypeStruct((B, O_dir, H, W), jnp.float32),
        ],
    )(
        x,
        W_cls, b_cls.reshape(O_cls, 1),
        W_reg, b_reg.reshape(O_reg, 1),
        W_dir, b_dir.reshape(O_dir, 1),
    )
    return outs
